# baseline (speedup 1.0000x reference)
import functools

import jax
import jax.numpy as jnp
from jax import lax
from jax.experimental import pallas as pl
from jax.experimental.pallas import tpu as pltpu
from jax.experimental.pallas import tpu_sc as plsc

N = 10000
E = 320000
D = 128

NC = 2
NS = 16
NW = NC * NS
B = 128
GBLK = E // B
MAXB = GBLK // NW + 1
RPT = 632
NR = 3


def _sc_segment_sum(x, edge_index):
    mesh = plsc.VectorSubcoreMesh(core_axis_name="c", subcore_axis_name="s")

    @functools.partial(
        pl.kernel,
        out_type=jax.ShapeDtypeStruct((NC * N, D), jnp.float32),
        mesh=mesh,
        scratch_types=dict(
            acc=pltpu.VMEM_SHARED((N, D), jnp.float32),
            sidx=pltpu.VMEM((4, B), jnp.int32),
            didx=pltpu.VMEM((4, B), jnp.int32),
            rows=pltpu.VMEM((NR, B, D), jnp.float32),
            isem=pltpu.SemaphoreType.DMA,
            gsem=pltpu.SemaphoreType.DMA,
            ssem=pltpu.SemaphoreType.DMA,
        ),
    )
    def seg_sum(x_hbm, e_hbm, out_hbm, acc, sidx, didx, rows,
                isem, gsem, ssem):
        c = lax.axis_index("c")
        s = lax.axis_index("s")
        wid = c * NS + s
        nblk = GBLK // NW + jnp.where(wid < GBLK % NW, 1, 0)

        z = jnp.zeros((16,), jnp.float32)

        def zrow(i, carry):
            for g in range(D // 16):
                rows[0, i, pl.ds(g * 16, 16)] = z
            return carry
        lax.fori_loop(0, B, zrow, 0)

        for k in range(RPT // B):
            pltpu.sync_copy(rows.at[0], acc.at[pl.ds(s * RPT + k * B, B)])

        @pl.when(s < NS - 1)
        def _():
            pltpu.sync_copy(
                rows.at[0, pl.ds(0, RPT % B)],
                acc.at[pl.ds(s * RPT + (RPT // B) * B, RPT % B)])

        @pl.when(s == NS - 1)
        def _():
            rem = N - (NS - 1) * RPT - (RPT // B) * B
            pltpu.sync_copy(
                rows.at[0, pl.ds(0, rem)],
                acc.at[pl.ds(s * RPT + (RPT // B) * B, rem)])

        plsc.subcore_barrier()

        pltpu.sync_copy(e_hbm.at[1, pl.ds(wid * B, B)], sidx.at[0])
        pltpu.sync_copy(e_hbm.at[0, pl.ds(wid * B, B)], didx.at[0])
        pltpu.async_copy(x_hbm.at[sidx.at[0]], rows.at[0], gsem)

        @pl.when(1 < nblk)
        def _():
            g1 = NW + wid
            pltpu.async_copy(e_hbm.at[1, pl.ds(g1 * B, B)], sidx.at[1], isem)
            pltpu.async_copy(e_hbm.at[0, pl.ds(g1 * B, B)], didx.at[1], isem)

        def edge_body(m, carry):
            b0 = lax.rem(m, NR)
            b1 = lax.rem(m + 1, NR)
            i4 = lax.rem(m, 4)

            @pl.when(m + 1 < nblk)
            def _():
                pltpu.make_async_copy(
                    e_hbm.at[1, pl.ds(0, B)], sidx.at[0], isem).wait()
                pltpu.make_async_copy(
                    e_hbm.at[0, pl.ds(0, B)], didx.at[0], isem).wait()

            @pl.when(jnp.logical_and(m >= 2, m - 2 < nblk))
            def _():
                pltpu.make_async_copy(
                    rows.at[0], acc.at[pl.ds(0, B)], ssem).wait()

            @pl.when(m + 1 < nblk)
            def _():
                pltpu.async_copy(x_hbm.at[sidx.at[lax.rem(m + 1, 4)]],
                                 rows.at[b1], gsem)

            @pl.when(m < nblk)
            def _():
                pltpu.make_async_copy(
                    x_hbm.at[sidx.at[i4]], rows.at[b0], gsem).wait()

            @pl.when(m + 2 < nblk)
            def _():
                p4 = lax.rem(m + 2, 4)
                g2 = (m + 2) * NW + wid
                pltpu.async_copy(e_hbm.at[1, pl.ds(g2 * B, B)],
                                 sidx.at[p4], isem)
                pltpu.async_copy(e_hbm.at[0, pl.ds(g2 * B, B)],
                                 didx.at[p4], isem)

            @pl.when(m < nblk)
            def _():
                pltpu.async_copy(rows.at[b0], acc.at[didx.at[i4]], ssem,
                                 add=True)
            return carry
        lax.fori_loop(0, MAXB, edge_body, 0)

        pltpu.make_async_copy(rows.at[0], acc.at[pl.ds(0, B)], ssem).wait()

        @pl.when(wid < GBLK % NW)
        def _():
            pltpu.make_async_copy(
                rows.at[0], acc.at[pl.ds(0, B)], ssem).wait()

        plsc.subcore_barrier()

        @pl.when(s < NS - 1)
        def _():
            pltpu.sync_copy(acc.at[pl.ds(s * RPT, RPT)],
                            out_hbm.at[pl.ds(c * N + s * RPT, RPT)])

        @pl.when(s == NS - 1)
        def _():
            last = N - (NS - 1) * RPT
            pltpu.sync_copy(acc.at[pl.ds(s * RPT, last)],
                            out_hbm.at[pl.ds(c * N + s * RPT, last)])

    return seg_sum(x, edge_index)


def _tc_linear(partials, W):
    BN = 2000

    def body(p_ref, w_ref, o_ref):
        agg = p_ref[0] + p_ref[1]
        o_ref[...] = lax.dot_general(
            agg, w_ref[...], (((1,), (1,)), ((), ())),
            preferred_element_type=jnp.float32)

    return pl.pallas_call(
        body,
        grid=(N // BN,),
        in_specs=[
            pl.BlockSpec((2, BN, D), lambda i: (0, i, 0)),
            pl.BlockSpec((D, D), lambda i: (0, 0)),
        ],
        out_specs=pl.BlockSpec((BN, D), lambda i: (i, 0)),
        out_shape=jax.ShapeDtypeStruct((N, D), jnp.float32),
    )(partials.reshape(2, N, D), W)


def kernel(x, edge_index, W):
    partials = _sc_segment_sum(x, edge_index)
    return _tc_linear(partials, W)


# speedup vs baseline: 1.0109x; 1.0109x over previous
import functools

import jax
import jax.numpy as jnp
from jax import lax
from jax.experimental import pallas as pl
from jax.experimental.pallas import tpu as pltpu
from jax.experimental.pallas import tpu_sc as plsc

N = 10000
E = 320000
D = 128

NC = 2
NS = 16
NW = NC * NS
B = 128
GBLK = E // B
MAXB = GBLK // NW + 1
RPT = 632
NR = 3


def _sc_segment_sum(x, edge_index):
    mesh = plsc.VectorSubcoreMesh(core_axis_name="c", subcore_axis_name="s")

    @functools.partial(
        pl.kernel,
        out_type=jax.ShapeDtypeStruct((NC * N, D), jnp.float32),
        mesh=mesh,
        scratch_types=dict(
            acc=pltpu.VMEM_SHARED((N, D), jnp.float32),
            sidx=pltpu.VMEM((4, B), jnp.int32),
            didx=pltpu.VMEM((4, B), jnp.int32),
            rows=pltpu.VMEM((NR, B, D), jnp.float32),
            isem=pltpu.SemaphoreType.DMA,
            gsem=pltpu.SemaphoreType.DMA,
            ssem=pltpu.SemaphoreType.DMA,
        ),
    )
    def seg_sum(x_hbm, e_hbm, out_hbm, acc, sidx, didx, rows,
                isem, gsem, ssem):
        c = lax.axis_index("c")
        s = lax.axis_index("s")
        wid = c * NS + s
        nblk = GBLK // NW + jnp.where(wid < GBLK % NW, 1, 0)

        z = jnp.zeros((16,), jnp.float32)

        def zrow(i, carry):
            for g in range(D // 16):
                rows[0, i, pl.ds(g * 16, 16)] = z
            return carry
        lax.fori_loop(0, B, zrow, 0)

        for k in range(RPT // B):
            pltpu.sync_copy(rows.at[0], acc.at[pl.ds(s * RPT + k * B, B)])

        @pl.when(s < NS - 1)
        def _():
            pltpu.sync_copy(
                rows.at[0, pl.ds(0, RPT % B)],
                acc.at[pl.ds(s * RPT + (RPT // B) * B, RPT % B)])

        @pl.when(s == NS - 1)
        def _():
            rem = N - (NS - 1) * RPT - (RPT // B) * B
            pltpu.sync_copy(
                rows.at[0, pl.ds(0, rem)],
                acc.at[pl.ds(s * RPT + (RPT // B) * B, rem)])

        pltpu.sync_copy(e_hbm.at[1, pl.ds(wid * B, B)], sidx.at[0])
        pltpu.sync_copy(e_hbm.at[0, pl.ds(wid * B, B)], didx.at[0])
        pltpu.async_copy(x_hbm.at[sidx.at[0]], rows.at[0], gsem)

        @pl.when(1 < nblk)
        def _():
            g1 = NW + wid
            pltpu.async_copy(e_hbm.at[1, pl.ds(g1 * B, B)], sidx.at[1], isem)
            pltpu.async_copy(e_hbm.at[0, pl.ds(g1 * B, B)], didx.at[1], isem)

        plsc.subcore_barrier()

        def edge_body(m, carry):
            b0 = lax.rem(m, NR)
            b1 = lax.rem(m + 1, NR)
            i4 = lax.rem(m, 4)

            @pl.when(m + 1 < nblk)
            def _():
                pltpu.make_async_copy(
                    e_hbm.at[1, pl.ds(0, B)], sidx.at[0], isem).wait()
                pltpu.make_async_copy(
                    e_hbm.at[0, pl.ds(0, B)], didx.at[0], isem).wait()

            @pl.when(jnp.logical_and(m >= 2, m - 2 < nblk))
            def _():
                pltpu.make_async_copy(
                    rows.at[0], acc.at[pl.ds(0, B)], ssem).wait()

            @pl.when(m + 1 < nblk)
            def _():
                pltpu.async_copy(x_hbm.at[sidx.at[lax.rem(m + 1, 4)]],
                                 rows.at[b1], gsem)

            @pl.when(m < nblk)
            def _():
                pltpu.make_async_copy(
                    x_hbm.at[sidx.at[i4]], rows.at[b0], gsem).wait()

            @pl.when(m + 2 < nblk)
            def _():
                p4 = lax.rem(m + 2, 4)
                g2 = (m + 2) * NW + wid
                pltpu.async_copy(e_hbm.at[1, pl.ds(g2 * B, B)],
                                 sidx.at[p4], isem)
                pltpu.async_copy(e_hbm.at[0, pl.ds(g2 * B, B)],
                                 didx.at[p4], isem)

            @pl.when(m < nblk)
            def _():
                pltpu.async_copy(rows.at[b0], acc.at[didx.at[i4]], ssem,
                                 add=True)
            return carry
        lax.fori_loop(0, MAXB, edge_body, 0)

        pltpu.make_async_copy(rows.at[0], acc.at[pl.ds(0, B)], ssem).wait()

        @pl.when(wid < GBLK % NW)
        def _():
            pltpu.make_async_copy(
                rows.at[0], acc.at[pl.ds(0, B)], ssem).wait()

        plsc.subcore_barrier()

        @pl.when(s < NS - 1)
        def _():
            pltpu.sync_copy(acc.at[pl.ds(s * RPT, RPT)],
                            out_hbm.at[pl.ds(c * N + s * RPT, RPT)])

        @pl.when(s == NS - 1)
        def _():
            last = N - (NS - 1) * RPT
            pltpu.sync_copy(acc.at[pl.ds(s * RPT, last)],
                            out_hbm.at[pl.ds(c * N + s * RPT, last)])

    return seg_sum(x, edge_index)


def _tc_linear(partials, W):
    BN = N

    def body(p_ref, w_ref, o_ref):
        agg = p_ref[0] + p_ref[1]
        o_ref[...] = lax.dot_general(
            agg, w_ref[...], (((1,), (1,)), ((), ())),
            preferred_element_type=jnp.float32)

    return pl.pallas_call(
        body,
        grid=(N // BN,),
        in_specs=[
            pl.BlockSpec((2, BN, D), lambda i: (0, i, 0)),
            pl.BlockSpec((D, D), lambda i: (0, 0)),
        ],
        out_specs=pl.BlockSpec((BN, D), lambda i: (i, 0)),
        out_shape=jax.ShapeDtypeStruct((N, D), jnp.float32),
    )(partials.reshape(2, N, D), W)


def kernel(x, edge_index, W):
    partials = _sc_segment_sum(x, edge_index)
    return _tc_linear(partials, W)


# speedup vs baseline: 1.1110x; 1.0990x over previous
import functools

import jax
import jax.numpy as jnp
from jax import lax
from jax.experimental import pallas as pl
from jax.experimental.pallas import tpu as pltpu
from jax.experimental.pallas import tpu_sc as plsc

N = 10000
E = 320000
D = 128

NC = 2
NS = 16
NW = NC * NS
B = 128
GBLK = E // B
MAXB = GBLK // NW + 1
RPT = 632
NR = 3


def _sc_segment_sum(x, edge_index):
    mesh = plsc.VectorSubcoreMesh(core_axis_name="c", subcore_axis_name="s")

    @functools.partial(
        pl.kernel,
        out_type=jax.ShapeDtypeStruct((NC * N, D), jnp.float32),
        mesh=mesh,
        scratch_types=dict(
            acc=pltpu.VMEM_SHARED((N, D), jnp.float32),
            sidx=pltpu.VMEM((4, B), jnp.int32),
            didx=pltpu.VMEM((4, B), jnp.int32),
            rows=pltpu.VMEM((NR, B, D), jnp.float32),
            isem=pltpu.SemaphoreType.DMA,
            gsem=pltpu.SemaphoreType.DMA,
            ssem=pltpu.SemaphoreType.DMA,
        ),
    )
    def seg_sum(x_hbm, e_hbm, out_hbm, acc, sidx, didx, rows,
                isem, gsem, ssem):
        c = lax.axis_index("c")
        s = lax.axis_index("s")
        wid = c * NS + s
        nblk = GBLK // NW + jnp.where(wid < GBLK % NW, 1, 0)

        z = jnp.zeros((16,), jnp.float32)

        def zrow(i, carry):
            for g in range(D // 16):
                rows[0, i, pl.ds(g * 16, 16)] = z
            return carry
        lax.fori_loop(0, B, zrow, 0)

        for k in range(RPT // B):
            pltpu.sync_copy(rows.at[0], acc.at[pl.ds(s * RPT + k * B, B)])

        @pl.when(s < NS - 1)
        def _():
            pltpu.sync_copy(
                rows.at[0, pl.ds(0, RPT % B)],
                acc.at[pl.ds(s * RPT + (RPT // B) * B, RPT % B)])

        @pl.when(s == NS - 1)
        def _():
            rem = N - (NS - 1) * RPT - (RPT // B) * B
            pltpu.sync_copy(
                rows.at[0, pl.ds(0, rem)],
                acc.at[pl.ds(s * RPT + (RPT // B) * B, rem)])

        pltpu.sync_copy(e_hbm.at[1, pl.ds(wid * B, B)], sidx.at[0])
        pltpu.sync_copy(e_hbm.at[0, pl.ds(wid * B, B)], didx.at[0])
        pltpu.async_copy(x_hbm.at[sidx.at[0]], rows.at[0], gsem)

        @pl.when(1 < nblk)
        def _():
            g1 = NW + wid
            pltpu.async_copy(e_hbm.at[1, pl.ds(g1 * B, B)], sidx.at[1], isem)
            pltpu.async_copy(e_hbm.at[0, pl.ds(g1 * B, B)], didx.at[1], isem)

        plsc.subcore_barrier()

        def edge_body(m, carry):
            b0 = lax.rem(m, NR)
            b1 = lax.rem(m + 1, NR)
            i4 = lax.rem(m, 4)

            @pl.when(m + 1 < nblk)
            def _():
                pltpu.make_async_copy(
                    e_hbm.at[1, pl.ds(0, B)], sidx.at[0], isem).wait()
                pltpu.make_async_copy(
                    e_hbm.at[0, pl.ds(0, B)], didx.at[0], isem).wait()

            @pl.when(jnp.logical_and(m >= 2, m - 2 < nblk))
            def _():
                pltpu.make_async_copy(
                    rows.at[0], acc.at[pl.ds(0, B)], ssem).wait()

            @pl.when(m + 1 < nblk)
            def _():
                pltpu.async_copy(x_hbm.at[sidx.at[lax.rem(m + 1, 4)]],
                                 rows.at[b1], gsem)

            @pl.when(m + 2 < nblk)
            def _():
                p4 = lax.rem(m + 2, 4)
                g2 = (m + 2) * NW + wid
                pltpu.async_copy(e_hbm.at[1, pl.ds(g2 * B, B)],
                                 sidx.at[p4], isem)
                pltpu.async_copy(e_hbm.at[0, pl.ds(g2 * B, B)],
                                 didx.at[p4], isem)

            @pl.when(m < nblk)
            def _():
                pltpu.make_async_copy(
                    x_hbm.at[sidx.at[i4]], rows.at[b0], gsem).wait()

            @pl.when(m < nblk)
            def _():
                pltpu.async_copy(rows.at[b0], acc.at[didx.at[i4]], ssem,
                                 add=True)
            return carry
        lax.fori_loop(0, MAXB, edge_body, 0)

        pltpu.make_async_copy(rows.at[0], acc.at[pl.ds(0, B)], ssem).wait()

        @pl.when(wid < GBLK % NW)
        def _():
            pltpu.make_async_copy(
                rows.at[0], acc.at[pl.ds(0, B)], ssem).wait()

        plsc.subcore_barrier()

        @pl.when(s < NS - 1)
        def _():
            pltpu.sync_copy(acc.at[pl.ds(s * RPT, RPT)],
                            out_hbm.at[pl.ds(c * N + s * RPT, RPT)])

        @pl.when(s == NS - 1)
        def _():
            last = N - (NS - 1) * RPT
            pltpu.sync_copy(acc.at[pl.ds(s * RPT, last)],
                            out_hbm.at[pl.ds(c * N + s * RPT, last)])

    return seg_sum(x, edge_index)


def _tc_linear(partials, W):
    BN = N

    def body(p_ref, w_ref, o_ref):
        agg = p_ref[0] + p_ref[1]
        o_ref[...] = lax.dot_general(
            agg, w_ref[...], (((1,), (1,)), ((), ())),
            preferred_element_type=jnp.float32)

    return pl.pallas_call(
        body,
        grid=(N // BN,),
        in_specs=[
            pl.BlockSpec((2, BN, D), lambda i: (0, i, 0)),
            pl.BlockSpec((D, D), lambda i: (0, 0)),
        ],
        out_specs=pl.BlockSpec((BN, D), lambda i: (i, 0)),
        out_shape=jax.ShapeDtypeStruct((N, D), jnp.float32),
    )(partials.reshape(2, N, D), W)


def kernel(x, edge_index, W):
    partials = _sc_segment_sum(x, edge_index)
    return _tc_linear(partials, W)
